# baseline (speedup 1.0000x reference)
import functools
import numpy as np
import jax
import jax.numpy as jnp
from jax import lax
from jax.experimental import pallas as pl
from jax.experimental.pallas import tpu as pltpu
from jax.experimental.pallas import tpu_sc as plsc

BANK = 8
BS = 32
NI = 300
D = 256
NPAD = 304
NV = NPAD // 16
CH = 16
NCHUNK = NPAD // CH
LANES = 16
DC = D // LANES
PL_UNROLL = 1


def _pe_table():
    inv_freq = (1.0 / (10000.0 ** (np.arange(0, D, 2, dtype=np.float32) / np.float32(D)))).astype(np.float32)
    pos = np.arange(BANK, dtype=np.float32)
    sin_inp = pos[:, None] * inv_freq[None, :]
    emb = np.stack((np.sin(sin_inp), np.cos(sin_inp)), axis=-1).reshape(BANK, D)
    return jnp.asarray(emb, dtype=jnp.float32)


_buf = lambda: [
    pltpu.VMEM((CH,), jnp.int32),
    pltpu.VMEM((CH,), jnp.int32),
    pltpu.VMEM((CH,), jnp.int32),
    pltpu.VMEM((BANK, CH, D), jnp.float32),
    pltpu.VMEM((CH, D), jnp.float32),
    pltpu.SemaphoreType.DMA,
]


@functools.partial(
    pl.kernel,
    mesh=plsc.VectorSubcoreMesh(core_axis_name="c", subcore_axis_name="s"),
    compiler_params=pltpu.CompilerParams(needs_layout_passes=False),
    out_type=jax.ShapeDtypeStruct((BS, NPAD, D), jnp.float32),
    scratch_types=[
        pltpu.VMEM((NPAD,), jnp.int32),
        pltpu.VMEM((NPAD,), jnp.int32),
        pltpu.VMEM((CH, D), jnp.float32),
        pltpu.VMEM((CH, D), jnp.float32),
        pltpu.SemaphoreType.DMA,
        pltpu.VMEM((BANK, D), jnp.float32),
    ] + _buf() + _buf(),
)
def _attn(mb_hbm, emb_hbm, q_hbm, ids_hbm, pe_hbm, out_hbm,
          ids_v, last_v, oA, oB, sem_o, pe_v,
          ii0, il0, iq0, g0, q0, sem0,
          ii1, il1, iq1, g1, q1, sem1):
    bufs = [(ii0, il0, iq0, g0, q0, sem0), (ii1, il1, iq1, g1, q1, sem1)]
    b = lax.axis_index("s") * 2 + lax.axis_index("c")

    pltpu.sync_copy(ids_hbm.at[b], ids_v)
    pltpu.sync_copy(pe_hbm, pe_v)

    iot = lax.iota(jnp.int32, LANES)
    zeros = jnp.zeros((LANES,), jnp.int32)

    for v in range(NV):
        last_v[pl.ds(v * LANES, LANES)] = zeros

    for v in range(NV):
        idvec = ids_v[pl.ds(v * LANES, LANES)]
        ivec = iot + (v * LANES)
        plsc.store_scatter(last_v, [idvec], ivec, mask=ivec < NI)

    drain_src = mb_hbm.at[0, 0, pl.ds(0, CH)]

    def fire(t, buf):
        ii, il, iq, g, q, sem = buf
        n0 = t * CH
        idvec = ids_v[pl.ds(n0, CH)]
        ii[...] = idvec
        il[...] = plsc.load_gather(last_v, [idvec])
        iq[...] = jnp.minimum(n0 + iot, NI - 1)
        for k in range(BANK - 1):
            pltpu.async_copy(mb_hbm.at[k, b].at[ii], g.at[k], sem)
        pltpu.async_copy(emb_hbm.at[b].at[il], g.at[BANK - 1], sem)
        pltpu.async_copy(q_hbm.at[b].at[iq], q, sem)

    def drain(buf):
        _, _, _, g, q, sem = buf
        for k in range(BANK):
            pltpu.make_async_copy(drain_src, g.at[k], sem).wait()
        pltpu.make_async_copy(drain_src, q, sem).wait()

    def drain_store(o_v):
        pltpu.make_async_copy(drain_src, o_v, sem_o).wait()

    def compute(t, buf, o_v):
        _, _, _, g_v, q_v, _ = buf

        for k in range(BANK):
            pek = [pe_v[k, pl.ds(c * LANES, LANES)] for c in range(DC)]

            @plsc.parallel_loop(0, CH, unroll=PL_UNROLL)
            def _pe_row(n, k=k, pek=pek):
                for c in range(DC):
                    sl = pl.ds(c * LANES, LANES)
                    g_v[k, n, sl] = g_v[k, n, sl] + pek[c]

        @plsc.parallel_loop(0, CH, unroll=PL_UNROLL)
        def _row(n):
            qr = [q_v[n, pl.ds(c * LANES, LANES)] for c in range(DC)]
            s = jnp.full((LANES,), -1e30, jnp.float32)
            for k in range(BANK):
                sl = pl.ds(0, LANES)
                acc = g_v[k, n, sl] * qr[0]
                for c in range(1, DC):
                    sl = pl.ds(c * LANES, LANES)
                    acc = acc + g_v[k, n, sl] * qr[c]
                s = jnp.where(iot == k, jnp.sum(acc), s)
            s = s * jnp.float32(1.0 / 16.0)
            e = jnp.exp(s - jnp.max(s))
            w = e / jnp.sum(e)
            wk = [w[k] for k in range(BANK)]
            for c in range(DC):
                sl = pl.ds(c * LANES, LANES)
                o = wk[0] * g_v[0, n, sl]
                for k in range(1, BANK):
                    o = o + wk[k] * g_v[k, n, sl]
                o_v[n, sl] = o
        pltpu.async_copy(o_v, out_hbm.at[b, pl.ds(t * CH, CH)], sem_o)

    fire(0, bufs[0])

    def _pair(t2, carry):
        t = t2 * 2
        fire(t + 1, bufs[1])
        drain(bufs[0])
        pl.when(t2 > 0)(lambda: drain_store(oA))
        compute(t, bufs[0], oA)
        fire(t + 2, bufs[0])
        drain(bufs[1])
        pl.when(t2 > 0)(lambda: drain_store(oB))
        compute(t + 1, bufs[1], oB)
        return carry

    lax.fori_loop(0, (NCHUNK - 1) // 2, _pair, 0)
    drain(bufs[0])
    drain_store(oA)
    compute(NCHUNK - 1, bufs[0], oA)
    drain_store(oA)
    drain_store(oB)


def kernel(mem_bank, embeddings, queries, mem_ids):
    ids = mem_ids.astype(jnp.int32)
    ids_pad = jnp.concatenate(
        [ids, jnp.zeros((BS, NPAD - NI), jnp.int32)], axis=1)
    out = _attn(mem_bank, embeddings, queries, ids_pad, _pe_table())
    return out[:, :NI, :]


# speedup vs baseline: 1.1598x; 1.1598x over previous
import functools
import numpy as np
import jax
import jax.numpy as jnp
from jax import lax
from jax.experimental import pallas as pl
from jax.experimental.pallas import tpu as pltpu
from jax.experimental.pallas import tpu_sc as plsc

BANK = 8
BS = 32
NI = 300
D = 256
NPAD = 304
NV = NPAD // 16
CH = 16
NCHUNK = NPAD // CH
LANES = 16
DC = D // LANES
PL_UNROLL = 2


def _pe_table():
    inv_freq = (1.0 / (10000.0 ** (np.arange(0, D, 2, dtype=np.float32) / np.float32(D)))).astype(np.float32)
    pos = np.arange(BANK, dtype=np.float32)
    sin_inp = pos[:, None] * inv_freq[None, :]
    emb = np.stack((np.sin(sin_inp), np.cos(sin_inp)), axis=-1).reshape(BANK, D)
    return jnp.asarray(emb, dtype=jnp.float32)


_buf = lambda: [
    pltpu.VMEM((CH,), jnp.int32),
    pltpu.VMEM((CH,), jnp.int32),
    pltpu.VMEM((CH,), jnp.int32),
    pltpu.VMEM((BANK, CH, D), jnp.float32),
    pltpu.VMEM((CH, D), jnp.float32),
    pltpu.SemaphoreType.DMA,
]


@functools.partial(
    pl.kernel,
    mesh=plsc.VectorSubcoreMesh(core_axis_name="c", subcore_axis_name="s"),
    compiler_params=pltpu.CompilerParams(needs_layout_passes=False),
    out_type=jax.ShapeDtypeStruct((BS, NPAD, D), jnp.float32),
    scratch_types=[
        pltpu.VMEM((NPAD,), jnp.int32),
        pltpu.VMEM((NPAD,), jnp.int32),
        pltpu.VMEM((CH, D), jnp.float32),
        pltpu.VMEM((CH, D), jnp.float32),
        pltpu.SemaphoreType.DMA,
        pltpu.VMEM((BANK, D), jnp.float32),
    ] + _buf() + _buf(),
)
def _attn(mb_hbm, emb_hbm, q_hbm, ids_hbm, pe_hbm, out_hbm,
          ids_v, last_v, oA, oB, sem_o, pe_v,
          ii0, il0, iq0, g0, q0, sem0,
          ii1, il1, iq1, g1, q1, sem1):
    bufs = [(ii0, il0, iq0, g0, q0, sem0), (ii1, il1, iq1, g1, q1, sem1)]
    b = lax.axis_index("s") * 2 + lax.axis_index("c")

    pltpu.sync_copy(ids_hbm.at[b], ids_v)
    pltpu.sync_copy(pe_hbm, pe_v)

    iot = lax.iota(jnp.int32, LANES)
    zeros = jnp.zeros((LANES,), jnp.int32)

    for v in range(NV):
        last_v[pl.ds(v * LANES, LANES)] = zeros

    for v in range(NV):
        idvec = ids_v[pl.ds(v * LANES, LANES)]
        ivec = iot + (v * LANES)
        plsc.store_scatter(last_v, [idvec], ivec, mask=ivec < NI)

    drain_src = mb_hbm.at[0, 0, pl.ds(0, CH)]

    def fire(t, buf):
        ii, il, iq, g, q, sem = buf
        n0 = t * CH
        idvec = ids_v[pl.ds(n0, CH)]
        ii[...] = idvec
        il[...] = plsc.load_gather(last_v, [idvec])
        iq[...] = jnp.minimum(n0 + iot, NI - 1)
        for k in range(BANK - 1):
            pltpu.async_copy(mb_hbm.at[k, b].at[ii], g.at[k], sem)
        pltpu.async_copy(emb_hbm.at[b].at[il], g.at[BANK - 1], sem)
        pltpu.async_copy(q_hbm.at[b].at[iq], q, sem)

    def drain(buf):
        _, _, _, g, q, sem = buf
        for k in range(BANK):
            pltpu.make_async_copy(drain_src, g.at[k], sem).wait()
        pltpu.make_async_copy(drain_src, q, sem).wait()

    def drain_store(o_v):
        pltpu.make_async_copy(drain_src, o_v, sem_o).wait()

    def compute(t, buf, o_v):
        _, _, _, g_v, q_v, _ = buf

        for k in range(BANK):
            pek = [pe_v[k, pl.ds(c * LANES, LANES)] for c in range(DC)]

            @plsc.parallel_loop(0, CH, unroll=PL_UNROLL)
            def _pe_row(n, k=k, pek=pek):
                for c in range(DC):
                    sl = pl.ds(c * LANES, LANES)
                    g_v[k, n, sl] = g_v[k, n, sl] + pek[c]

        @plsc.parallel_loop(0, CH, unroll=PL_UNROLL)
        def _row(n):
            qr = [q_v[n, pl.ds(c * LANES, LANES)] for c in range(DC)]
            s = jnp.full((LANES,), -1e30, jnp.float32)
            for k in range(BANK):
                sl = pl.ds(0, LANES)
                acc = g_v[k, n, sl] * qr[0]
                for c in range(1, DC):
                    sl = pl.ds(c * LANES, LANES)
                    acc = acc + g_v[k, n, sl] * qr[c]
                s = jnp.where(iot == k, jnp.sum(acc), s)
            s = s * jnp.float32(1.0 / 16.0)
            e = jnp.exp(s - jnp.max(s))
            w = e / jnp.sum(e)
            wk = [w[k] for k in range(BANK)]
            for c in range(DC):
                sl = pl.ds(c * LANES, LANES)
                o = wk[0] * g_v[0, n, sl]
                for k in range(1, BANK):
                    o = o + wk[k] * g_v[k, n, sl]
                o_v[n, sl] = o
        pltpu.async_copy(o_v, out_hbm.at[b, pl.ds(t * CH, CH)], sem_o)

    fire(0, bufs[0])

    def _pair(t2, carry):
        t = t2 * 2
        fire(t + 1, bufs[1])
        drain(bufs[0])
        pl.when(t2 > 0)(lambda: drain_store(oA))
        compute(t, bufs[0], oA)
        fire(t + 2, bufs[0])
        drain(bufs[1])
        pl.when(t2 > 0)(lambda: drain_store(oB))
        compute(t + 1, bufs[1], oB)
        return carry

    lax.fori_loop(0, (NCHUNK - 1) // 2, _pair, 0)
    drain(bufs[0])
    drain_store(oA)
    compute(NCHUNK - 1, bufs[0], oA)
    drain_store(oA)
    drain_store(oB)


def kernel(mem_bank, embeddings, queries, mem_ids):
    ids = mem_ids.astype(jnp.int32)
    ids_pad = jnp.concatenate(
        [ids, jnp.zeros((BS, NPAD - NI), jnp.int32)], axis=1)
    out = _attn(mem_bank, embeddings, queries, ids_pad, _pe_table())
    return out[:, :NI, :]
